# baseline (speedup 1.0000x reference)
import functools

import jax
import jax.numpy as jnp
from jax import lax
from jax.experimental import pallas as pl
from jax.experimental.pallas import tpu as pltpu
from jax.experimental.pallas import tpu_sc as plsc

N = 10000
E = 320000
D = 128

_NC = 2
_NS = 16
_NW = _NC * _NS
_CHUNK = 128
_NCH = 80
_EPT = _NCH * _CHUNK
_EPAD = _NW * _EPT
_RPT = 632
_RPT_LAST = N - 15 * _RPT
_AGG_ROWS = N + _CHUNK
_SPAN = 40
_TSPAN = _SPAN // 2


def _rowcopy(src_ref, dst_ref, sid):
    @pl.when(sid < _NS - 1)
    def _():
        pltpu.sync_copy(src_ref.at[pl.ds(sid * _RPT, _RPT)],
                        dst_ref.at[pl.ds(sid * _RPT, _RPT)])

    @pl.when(sid == _NS - 1)
    def _():
        pltpu.sync_copy(src_ref.at[pl.ds(15 * _RPT, _RPT_LAST)],
                        dst_ref.at[pl.ds(15 * _RPT, _RPT_LAST)])


def _sc_agg_body(h_hbm, src_hbm, dst_hbm, zero_hbm, agg0_hbm, agg1_hbm,
                 agg_sh, idx_s, idx_d, rows0, rows1, sem0, sem1, ssem0, ssem1):
    cid = lax.axis_index("c")
    sid = lax.axis_index("s")
    wid = sid * _NC + cid

    @pl.when(cid == 0)
    def _():
        _rowcopy(h_hbm, agg_sh, sid)

    @pl.when(cid != 0)
    def _():
        _rowcopy(zero_hbm, agg_sh, sid)

    plsc.subcore_barrier()

    def span(base):
        pltpu.sync_copy(src_hbm.at[pl.ds(wid * _NCH + base, _SPAN)], idx_s)
        pltpu.sync_copy(dst_hbm.at[pl.ds(wid * _NCH + base, _SPAN)], idx_d)

        def it(c, carry):
            pltpu.async_copy(h_hbm.at[idx_s.at[c]], rows0, sem0)
            pltpu.make_async_copy(h_hbm.at[idx_s.at[c]], rows0, sem0).wait()
            pltpu.sync_copy(rows0, agg_sh.at[idx_d.at[c]], add=True)
            return carry

        lax.fori_loop(0, _SPAN, it, 0)

    span(0)
    span(_SPAN)

    plsc.subcore_barrier()

    @pl.when(cid == 0)
    def _():
        _rowcopy(agg_sh, agg0_hbm, sid)

    @pl.when(cid != 0)
    def _():
        _rowcopy(agg_sh, agg1_hbm, sid)


def _sc_agg(h, src2d, dst2d, zeros):
    mesh = plsc.VectorSubcoreMesh(core_axis_name="c", subcore_axis_name="s",
                                  num_cores=_NC, num_subcores=_NS)
    f = pl.kernel(
        _sc_agg_body,
        out_type=(jax.ShapeDtypeStruct((N, D), jnp.float32),
                  jax.ShapeDtypeStruct((N, D), jnp.float32)),
        mesh=mesh,
        scratch_types=[
            pltpu.VMEM_SHARED((_AGG_ROWS, D), jnp.float32),
            pltpu.VMEM((_SPAN, _CHUNK), jnp.int32),
            pltpu.VMEM((_SPAN, _CHUNK), jnp.int32),
            pltpu.VMEM((_CHUNK, D), jnp.float32),
            pltpu.VMEM((_CHUNK, D), jnp.float32),
            pltpu.SemaphoreType.DMA,
            pltpu.SemaphoreType.DMA,
            pltpu.SemaphoreType.DMA,
            pltpu.SemaphoreType.DMA,
        ],
    )
    return f(h, src2d, dst2d, zeros)


def _mlp_block(a0_ref, a1_ref, w1_ref, b1_ref, w2_ref, b2_ref, out_ref, *,
               out_relu):
    z = a0_ref[...] + a1_ref[...]
    z = jnp.maximum(
        jnp.dot(z, w1_ref[...], preferred_element_type=jnp.float32)
        + b1_ref[...], 0.0)
    o = jnp.dot(z, w2_ref[...], preferred_element_type=jnp.float32) + b2_ref[...]
    if out_relu:
        o = jnp.maximum(o, 0.0)
    out_ref[...] = o


def _mlp(a0, a1, w1, b1, w2, b2, out_relu):
    blk = 1000
    return pl.pallas_call(
        functools.partial(_mlp_block, out_relu=out_relu),
        grid=(N // blk,),
        in_specs=[
            pl.BlockSpec((blk, D), lambda i: (i, 0)),
            pl.BlockSpec((blk, D), lambda i: (i, 0)),
            pl.BlockSpec((D, D), lambda i: (0, 0)),
            pl.BlockSpec((1, D), lambda i: (0, 0)),
            pl.BlockSpec((D, D), lambda i: (0, 0)),
            pl.BlockSpec((1, D), lambda i: (0, 0)),
        ],
        out_specs=pl.BlockSpec((blk, D), lambda i: (i, 0)),
        out_shape=jax.ShapeDtypeStruct((N, D), jnp.float32),
    )(a0, a1, w1, b1.reshape(1, D), w2, b2.reshape(1, D))


def kernel(x, edge_index, W1_0, b1_0, W2_0, b2_0, W1_1, b1_1, W2_1, b2_1):
    src = edge_index[0]
    dst = edge_index[1]
    ept_real = E // _NW
    pad_per_tile = _EPT - ept_real
    pad_src = jnp.zeros((_NW, pad_per_tile), jnp.int32)
    pad_dst = jnp.broadcast_to(
        N + (jnp.arange(pad_per_tile, dtype=jnp.int32) % _CHUNK),
        (_NW, pad_per_tile))
    src2d = jnp.concatenate(
        [src.reshape(_NW, ept_real), pad_src], axis=1).reshape(
            _NW * _NCH, _CHUNK)
    dst2d = jnp.concatenate(
        [dst.reshape(_NW, ept_real), pad_dst], axis=1).reshape(
            _NW * _NCH, _CHUNK)
    zeros = jnp.zeros((N, D), jnp.float32)

    a0, a1 = _sc_agg(x, src2d, dst2d, zeros)
    h1 = _mlp(a0, a1, W1_0, b1_0, W2_0, b2_0, out_relu=True)
    a0, a1 = _sc_agg(h1, src2d, dst2d, zeros)
    return _mlp(a0, a1, W1_1, b1_1, W2_1, b2_1, out_relu=False)


# speedup vs baseline: 1.1502x; 1.1502x over previous
import functools

import jax
import jax.numpy as jnp
from jax import lax
from jax.experimental import pallas as pl
from jax.experimental.pallas import tpu as pltpu
from jax.experimental.pallas import tpu_sc as plsc

N = 10000
E = 320000
D = 128

_NC = 2
_NS = 16
_NW = _NC * _NS
_CHUNK = 128
_NCH = 80
_EPT = _NCH * _CHUNK
_EPAD = _NW * _EPT
_RPT = 632
_RPT_LAST = N - 15 * _RPT
_AGG_ROWS = N + _CHUNK
_SPAN = 40
_TSPAN = _SPAN // 2


def _rowcopy(src_ref, dst_ref, sid):
    @pl.when(sid < _NS - 1)
    def _():
        pltpu.sync_copy(src_ref.at[pl.ds(sid * _RPT, _RPT)],
                        dst_ref.at[pl.ds(sid * _RPT, _RPT)])

    @pl.when(sid == _NS - 1)
    def _():
        pltpu.sync_copy(src_ref.at[pl.ds(15 * _RPT, _RPT_LAST)],
                        dst_ref.at[pl.ds(15 * _RPT, _RPT_LAST)])


def _sc_agg_body(h_hbm, src_hbm, dst_hbm, zero_hbm, agg0_hbm, agg1_hbm,
                 agg_sh, idx_s, idx_d, rows0, rows1, sem0, sem1, ssem0, ssem1):
    cid = lax.axis_index("c")
    sid = lax.axis_index("s")
    wid = sid * _NC + cid

    @pl.when(cid == 0)
    def _():
        _rowcopy(h_hbm, agg_sh, sid)

    @pl.when(cid != 0)
    def _():
        _rowcopy(zero_hbm, agg_sh, sid)

    plsc.subcore_barrier()

    def span(base):
        pltpu.sync_copy(src_hbm.at[pl.ds(wid * _NCH + base, _SPAN)], idx_s)
        pltpu.sync_copy(dst_hbm.at[pl.ds(wid * _NCH + base, _SPAN)], idx_d)
        pltpu.async_copy(h_hbm.at[idx_s.at[0]], rows0, sem0)
        pltpu.async_copy(h_hbm.at[idx_s.at[1]], rows1, sem1)

        def it(t, carry):
            c0 = 2 * t
            c1 = c0 + 1
            pltpu.make_async_copy(h_hbm.at[idx_s.at[c0]], rows0, sem0).wait()
            pltpu.sync_copy(rows0, agg_sh.at[idx_d.at[c0]], add=True)

            @pl.when(t < _TSPAN - 1)
            def _():
                pltpu.async_copy(h_hbm.at[idx_s.at[c0 + 2]], rows0, sem0)

            pltpu.make_async_copy(h_hbm.at[idx_s.at[c1]], rows1, sem1).wait()
            pltpu.sync_copy(rows1, agg_sh.at[idx_d.at[c1]], add=True)

            @pl.when(t < _TSPAN - 1)
            def _():
                pltpu.async_copy(h_hbm.at[idx_s.at[c1 + 2]], rows1, sem1)

            return carry

        lax.fori_loop(0, _TSPAN, it, 0)

    span(0)
    span(_SPAN)

    plsc.subcore_barrier()

    @pl.when(cid == 0)
    def _():
        _rowcopy(agg_sh, agg0_hbm, sid)

    @pl.when(cid != 0)
    def _():
        _rowcopy(agg_sh, agg1_hbm, sid)


def _sc_agg(h, src2d, dst2d, zeros):
    mesh = plsc.VectorSubcoreMesh(core_axis_name="c", subcore_axis_name="s",
                                  num_cores=_NC, num_subcores=_NS)
    f = pl.kernel(
        _sc_agg_body,
        out_type=(jax.ShapeDtypeStruct((N, D), jnp.float32),
                  jax.ShapeDtypeStruct((N, D), jnp.float32)),
        mesh=mesh,
        scratch_types=[
            pltpu.VMEM_SHARED((_AGG_ROWS, D), jnp.float32),
            pltpu.VMEM((_SPAN, _CHUNK), jnp.int32),
            pltpu.VMEM((_SPAN, _CHUNK), jnp.int32),
            pltpu.VMEM((_CHUNK, D), jnp.float32),
            pltpu.VMEM((_CHUNK, D), jnp.float32),
            pltpu.SemaphoreType.DMA,
            pltpu.SemaphoreType.DMA,
            pltpu.SemaphoreType.DMA,
            pltpu.SemaphoreType.DMA,
        ],
    )
    return f(h, src2d, dst2d, zeros)


def _mlp_block(a0_ref, a1_ref, w1_ref, b1_ref, w2_ref, b2_ref, out_ref, *,
               out_relu):
    z = a0_ref[...] + a1_ref[...]
    z = jnp.maximum(
        jnp.dot(z, w1_ref[...], preferred_element_type=jnp.float32)
        + b1_ref[...], 0.0)
    o = jnp.dot(z, w2_ref[...], preferred_element_type=jnp.float32) + b2_ref[...]
    if out_relu:
        o = jnp.maximum(o, 0.0)
    out_ref[...] = o


def _mlp(a0, a1, w1, b1, w2, b2, out_relu):
    blk = 1000
    return pl.pallas_call(
        functools.partial(_mlp_block, out_relu=out_relu),
        grid=(N // blk,),
        in_specs=[
            pl.BlockSpec((blk, D), lambda i: (i, 0)),
            pl.BlockSpec((blk, D), lambda i: (i, 0)),
            pl.BlockSpec((D, D), lambda i: (0, 0)),
            pl.BlockSpec((1, D), lambda i: (0, 0)),
            pl.BlockSpec((D, D), lambda i: (0, 0)),
            pl.BlockSpec((1, D), lambda i: (0, 0)),
        ],
        out_specs=pl.BlockSpec((blk, D), lambda i: (i, 0)),
        out_shape=jax.ShapeDtypeStruct((N, D), jnp.float32),
    )(a0, a1, w1, b1.reshape(1, D), w2, b2.reshape(1, D))


def kernel(x, edge_index, W1_0, b1_0, W2_0, b2_0, W1_1, b1_1, W2_1, b2_1):
    src = edge_index[0]
    dst = edge_index[1]
    ept_real = E // _NW
    pad_per_tile = _EPT - ept_real
    pad_src = jnp.zeros((_NW, pad_per_tile), jnp.int32)
    pad_dst = jnp.broadcast_to(
        N + (jnp.arange(pad_per_tile, dtype=jnp.int32) % _CHUNK),
        (_NW, pad_per_tile))
    src2d = jnp.concatenate(
        [src.reshape(_NW, ept_real), pad_src], axis=1).reshape(
            _NW * _NCH, _CHUNK)
    dst2d = jnp.concatenate(
        [dst.reshape(_NW, ept_real), pad_dst], axis=1).reshape(
            _NW * _NCH, _CHUNK)
    zeros = jnp.zeros((N, D), jnp.float32)

    a0, a1 = _sc_agg(x, src2d, dst2d, zeros)
    h1 = _mlp(a0, a1, W1_0, b1_0, W2_0, b2_0, out_relu=True)
    a0, a1 = _sc_agg(h1, src2d, dst2d, zeros)
    return _mlp(a0, a1, W1_1, b1_1, W2_1, b2_1, out_relu=False)


# speedup vs baseline: 1.1536x; 1.0029x over previous
import functools

import jax
import jax.numpy as jnp
from jax import lax
from jax.experimental import pallas as pl
from jax.experimental.pallas import tpu as pltpu
from jax.experimental.pallas import tpu_sc as plsc

N = 10000
E = 320000
D = 128

_NC = 2
_NS = 16
_NW = _NC * _NS
_CHUNK = 128
_NCH = 80
_EPT = _NCH * _CHUNK
_EPAD = _NW * _EPT
_RPT = 632
_RPT_LAST = N - 15 * _RPT
_AGG_ROWS = N + _CHUNK
_SPAN = 40
_TSPAN = _SPAN // 2


def _rowcopy(src_ref, dst_ref, sid):
    @pl.when(sid < _NS - 1)
    def _():
        pltpu.sync_copy(src_ref.at[pl.ds(sid * _RPT, _RPT)],
                        dst_ref.at[pl.ds(sid * _RPT, _RPT)])

    @pl.when(sid == _NS - 1)
    def _():
        pltpu.sync_copy(src_ref.at[pl.ds(15 * _RPT, _RPT_LAST)],
                        dst_ref.at[pl.ds(15 * _RPT, _RPT_LAST)])


def _sc_agg_body(h_hbm, src_hbm, dst_hbm, zero_hbm, agg0_hbm, agg1_hbm,
                 agg_sh, idx_s, idx_d, rows0, rows1, sem0, sem1, ssem0, ssem1):
    cid = lax.axis_index("c")
    sid = lax.axis_index("s")
    wid = sid * _NC + cid

    pltpu.sync_copy(src_hbm.at[pl.ds(wid * _NCH, _SPAN)], idx_s)
    pltpu.sync_copy(dst_hbm.at[pl.ds(wid * _NCH, _SPAN)], idx_d)
    pltpu.async_copy(h_hbm.at[idx_s.at[0]], rows0, sem0)
    pltpu.async_copy(h_hbm.at[idx_s.at[1]], rows1, sem1)

    @pl.when(cid == 0)
    def _():
        _rowcopy(h_hbm, agg_sh, sid)

    @pl.when(cid != 0)
    def _():
        _rowcopy(zero_hbm, agg_sh, sid)

    plsc.subcore_barrier()

    def span(base, prefetched=False):
        if not prefetched:
            pltpu.sync_copy(src_hbm.at[pl.ds(wid * _NCH + base, _SPAN)],
                            idx_s)
            pltpu.sync_copy(dst_hbm.at[pl.ds(wid * _NCH + base, _SPAN)],
                            idx_d)
            pltpu.async_copy(h_hbm.at[idx_s.at[0]], rows0, sem0)
            pltpu.async_copy(h_hbm.at[idx_s.at[1]], rows1, sem1)

        def it(t, carry):
            c0 = 2 * t
            c1 = c0 + 1
            pltpu.make_async_copy(h_hbm.at[idx_s.at[c0]], rows0, sem0).wait()
            pltpu.sync_copy(rows0, agg_sh.at[idx_d.at[c0]], add=True)

            @pl.when(t < _TSPAN - 1)
            def _():
                pltpu.async_copy(h_hbm.at[idx_s.at[c0 + 2]], rows0, sem0)

            pltpu.make_async_copy(h_hbm.at[idx_s.at[c1]], rows1, sem1).wait()
            pltpu.sync_copy(rows1, agg_sh.at[idx_d.at[c1]], add=True)

            @pl.when(t < _TSPAN - 1)
            def _():
                pltpu.async_copy(h_hbm.at[idx_s.at[c1 + 2]], rows1, sem1)

            return carry

        lax.fori_loop(0, _TSPAN, it, 0)

    span(0, prefetched=True)
    span(_SPAN)

    plsc.subcore_barrier()

    @pl.when(cid == 0)
    def _():
        _rowcopy(agg_sh, agg0_hbm, sid)

    @pl.when(cid != 0)
    def _():
        _rowcopy(agg_sh, agg1_hbm, sid)


def _sc_agg(h, src2d, dst2d, zeros):
    mesh = plsc.VectorSubcoreMesh(core_axis_name="c", subcore_axis_name="s",
                                  num_cores=_NC, num_subcores=_NS)
    f = pl.kernel(
        _sc_agg_body,
        out_type=(jax.ShapeDtypeStruct((N, D), jnp.float32),
                  jax.ShapeDtypeStruct((N, D), jnp.float32)),
        mesh=mesh,
        scratch_types=[
            pltpu.VMEM_SHARED((_AGG_ROWS, D), jnp.float32),
            pltpu.VMEM((_SPAN, _CHUNK), jnp.int32),
            pltpu.VMEM((_SPAN, _CHUNK), jnp.int32),
            pltpu.VMEM((_CHUNK, D), jnp.float32),
            pltpu.VMEM((_CHUNK, D), jnp.float32),
            pltpu.SemaphoreType.DMA,
            pltpu.SemaphoreType.DMA,
            pltpu.SemaphoreType.DMA,
            pltpu.SemaphoreType.DMA,
        ],
    )
    return f(h, src2d, dst2d, zeros)


def _mlp_block(a0_ref, a1_ref, w1_ref, b1_ref, w2_ref, b2_ref, out_ref, *,
               out_relu):
    z = a0_ref[...] + a1_ref[...]
    z = jnp.maximum(
        jnp.dot(z, w1_ref[...], preferred_element_type=jnp.float32)
        + b1_ref[...], 0.0)
    o = jnp.dot(z, w2_ref[...], preferred_element_type=jnp.float32) + b2_ref[...]
    if out_relu:
        o = jnp.maximum(o, 0.0)
    out_ref[...] = o


def _mlp(a0, a1, w1, b1, w2, b2, out_relu):
    blk = 1000
    return pl.pallas_call(
        functools.partial(_mlp_block, out_relu=out_relu),
        grid=(N // blk,),
        in_specs=[
            pl.BlockSpec((blk, D), lambda i: (i, 0)),
            pl.BlockSpec((blk, D), lambda i: (i, 0)),
            pl.BlockSpec((D, D), lambda i: (0, 0)),
            pl.BlockSpec((1, D), lambda i: (0, 0)),
            pl.BlockSpec((D, D), lambda i: (0, 0)),
            pl.BlockSpec((1, D), lambda i: (0, 0)),
        ],
        out_specs=pl.BlockSpec((blk, D), lambda i: (i, 0)),
        out_shape=jax.ShapeDtypeStruct((N, D), jnp.float32),
    )(a0, a1, w1, b1.reshape(1, D), w2, b2.reshape(1, D))


def kernel(x, edge_index, W1_0, b1_0, W2_0, b2_0, W1_1, b1_1, W2_1, b2_1):
    src = edge_index[0]
    dst = edge_index[1]
    ept_real = E // _NW
    pad_per_tile = _EPT - ept_real
    pad_src = jnp.zeros((_NW, pad_per_tile), jnp.int32)
    pad_dst = jnp.broadcast_to(
        N + (jnp.arange(pad_per_tile, dtype=jnp.int32) % _CHUNK),
        (_NW, pad_per_tile))
    src2d = jnp.concatenate(
        [src.reshape(_NW, ept_real), pad_src], axis=1).reshape(
            _NW * _NCH, _CHUNK)
    dst2d = jnp.concatenate(
        [dst.reshape(_NW, ept_real), pad_dst], axis=1).reshape(
            _NW * _NCH, _CHUNK)
    zeros = jnp.zeros((N, D), jnp.float32)

    a0, a1 = _sc_agg(x, src2d, dst2d, zeros)
    h1 = _mlp(a0, a1, W1_0, b1_0, W2_0, b2_0, out_relu=True)
    a0, a1 = _sc_agg(h1, src2d, dst2d, zeros)
    return _mlp(a0, a1, W1_1, b1_1, W2_1, b2_1, out_relu=False)
